# baseline (device time: 323299 ns/iter reference)
import functools

import jax
import jax.numpy as jnp
from jax import lax
from jax.experimental import pallas as pl
from jax.experimental.pallas import tpu as pltpu

N_DEV = 4
B, SQ, H, D = 16, 1, 16, 64
SCALE = D ** -0.5


def _local_partials(Q, K, V):
    kv_per = K.shape[1]

    def body(q_ref, k_ref, v_ref, u_ref, m_ref, l_ref):
        q = q_ref[0, 0]
        k = k_ref[0]
        v = v_ref[0]
        ones_d = jnp.ones((D,), jnp.float32)
        s = lax.dot_general(
            k * q[None], ones_d,
            dimension_numbers=(((2,), (0,)), ((), ())),
            preferred_element_type=jnp.float32,
        ) * SCALE
        m = jnp.max(s, axis=0)
        p = jnp.exp(s - m[None])
        l = jnp.sum(p, axis=0)
        u = jnp.sum(v * p[:, :, None], axis=0)
        u_ref[0] = u
        m_ref[0, 0, :] = m
        l_ref[0, 0, :] = l

    return pl.pallas_call(
        body,
        grid=(B,),
        in_specs=[
            pl.BlockSpec((1, SQ, H, D), lambda b: (b, 0, 0, 0)),
            pl.BlockSpec((1, kv_per, H, D), lambda b: (b, 0, 0, 0)),
            pl.BlockSpec((1, kv_per, H, D), lambda b: (b, 0, 0, 0)),
        ],
        out_specs=[
            pl.BlockSpec((1, H, D), lambda b: (b, 0, 0)),
            pl.BlockSpec((1, 1, H), lambda b: (b, 0, 0)),
            pl.BlockSpec((1, 1, H), lambda b: (b, 0, 0)),
        ],
        out_shape=[
            jax.ShapeDtypeStruct((B, H, D), jnp.float32),
            jax.ShapeDtypeStruct((B, 1, H), jnp.float32),
            jax.ShapeDtypeStruct((B, 1, H), jnp.float32),
        ],
        compiler_params=pltpu.CompilerParams(
            vmem_limit_bytes=100 * 1024 * 1024,
        ),
    )(Q, K, V)


def _ring_combine(U, m, l):

    def body(u_ref, m_ref, l_ref, out_ref, g_u, g_ml, send_u, recv_u,
             send_ml, recv_ml):
        my = lax.axis_index("i")
        left = lax.rem(my + N_DEV - 1, N_DEV)
        right = lax.rem(my + 1, N_DEV)

        barrier = pltpu.get_barrier_semaphore()
        for nbr in (left, right):
            pl.semaphore_signal(
                barrier, inc=1, device_id=(nbr,),
                device_id_type=pl.DeviceIdType.MESH,
            )
        pl.semaphore_wait(barrier, 2)

        g_u[pl.ds(my, 1)] = u_ref[...][None]
        g_ml[pl.ds(my, 1), 0] = m_ref[:, 0, :][None]
        g_ml[pl.ds(my, 1), 1] = l_ref[:, 0, :][None]

        for h in range(N_DEV - 1):
            slot = lax.rem(my + N_DEV - h, N_DEV)
            rdma_u = pltpu.make_async_remote_copy(
                src_ref=g_u.at[pl.ds(slot, 1)],
                dst_ref=g_u.at[pl.ds(slot, 1)],
                send_sem=send_u.at[h],
                recv_sem=recv_u.at[h],
                device_id=(right,),
                device_id_type=pl.DeviceIdType.MESH,
            )
            rdma_ml = pltpu.make_async_remote_copy(
                src_ref=g_ml.at[pl.ds(slot, 1)],
                dst_ref=g_ml.at[pl.ds(slot, 1)],
                send_sem=send_ml.at[h],
                recv_sem=recv_ml.at[h],
                device_id=(right,),
                device_id_type=pl.DeviceIdType.MESH,
            )
            rdma_u.start()
            rdma_ml.start()
            rdma_u.wait()
            rdma_ml.wait()

        m_all = g_ml[:, 0]
        l_all = g_ml[:, 1]
        m_g = jnp.max(m_all, axis=0)
        alpha = jnp.exp(m_all - m_g[None])
        l_g = jnp.sum(l_all * alpha, axis=0)
        u_g = jnp.sum(g_u[...] * alpha[..., None], axis=0)
        out_ref[...] = (u_g / l_g[..., None])[:, None]

    return pl.pallas_call(
        body,
        out_shape=jax.ShapeDtypeStruct((B, SQ, H, D), jnp.float32),
        in_specs=[pl.BlockSpec(memory_space=pltpu.VMEM)] * 3,
        out_specs=pl.BlockSpec(memory_space=pltpu.VMEM),
        scratch_shapes=[
            pltpu.VMEM((N_DEV, B, H, D), jnp.float32),
            pltpu.VMEM((N_DEV, 2, B, H), jnp.float32),
            pltpu.SemaphoreType.DMA((N_DEV - 1,)),
            pltpu.SemaphoreType.DMA((N_DEV - 1,)),
            pltpu.SemaphoreType.DMA((N_DEV - 1,)),
            pltpu.SemaphoreType.DMA((N_DEV - 1,)),
        ],
        compiler_params=pltpu.CompilerParams(collective_id=0),
    )(U, m, l)


def kernel(Q, K, V):
    U, m, l = _local_partials(Q, K, V)
    return _ring_combine(U, m, l)


# device time: 58296 ns/iter; 5.5458x vs baseline; 5.5458x over previous
import functools

import jax
import jax.numpy as jnp
from jax import lax
from jax.experimental import pallas as pl
from jax.experimental.pallas import tpu as pltpu

N_DEV = 4
B, SQ, H, D = 16, 1, 16, 64
SCALE = D ** -0.5


def _local_partials(Q, Kt, Vt):
    kv_per = Kt.shape[3]
    hd = H * D

    def body(q_ref, k_ref, v_ref, u_ref, m_ref, l_ref):
        q = q_ref[0, 0]
        k2 = k_ref[0].reshape(hd, kv_per)
        v2 = v_ref[0].reshape(hd, kv_per)
        j = lax.broadcasted_iota(jnp.int32, (H, hd), 1)
        hh = lax.broadcasted_iota(jnp.int32, (H, hd), 0)
        A = jnp.where(j // D == hh, jnp.tile(q, (1, H)), 0.0)
        s = lax.dot_general(
            A, k2,
            dimension_numbers=(((1,), (0,)), ((), ())),
            preferred_element_type=jnp.float32,
        ) * SCALE
        m = jnp.max(s, axis=1)
        p = jnp.exp(s - m[:, None])
        l = jnp.sum(p, axis=1)
        p2 = jnp.broadcast_to(p[:, None, :], (H, D, kv_per)).reshape(hd, kv_per)
        ones_kv = jnp.ones((kv_per,), jnp.float32)
        u = lax.dot_general(
            v2 * p2, ones_kv,
            dimension_numbers=(((1,), (0,)), ((), ())),
            preferred_element_type=jnp.float32,
        ).reshape(H, D)
        u_ref[0] = u
        m_ref[0, 0, :] = m
        l_ref[0, 0, :] = l

    return pl.pallas_call(
        body,
        grid=(B,),
        in_specs=[
            pl.BlockSpec((1, SQ, H, D), lambda b: (b, 0, 0, 0)),
            pl.BlockSpec((1, H, D, kv_per), lambda b: (b, 0, 0, 0)),
            pl.BlockSpec((1, H, D, kv_per), lambda b: (b, 0, 0, 0)),
        ],
        out_specs=[
            pl.BlockSpec((1, H, D), lambda b: (b, 0, 0)),
            pl.BlockSpec((1, 1, H), lambda b: (b, 0, 0)),
            pl.BlockSpec((1, 1, H), lambda b: (b, 0, 0)),
        ],
        out_shape=[
            jax.ShapeDtypeStruct((B, H, D), jnp.float32),
            jax.ShapeDtypeStruct((B, 1, H), jnp.float32),
            jax.ShapeDtypeStruct((B, 1, H), jnp.float32),
        ],
        compiler_params=pltpu.CompilerParams(
            vmem_limit_bytes=100 * 1024 * 1024,
        ),
    )(Q, Kt, Vt)


def _ring_combine(U, m, l):

    def body(u_ref, m_ref, l_ref, out_ref, g_u, g_ml, send_u, recv_u,
             send_ml, recv_ml):
        my = lax.axis_index("i")
        left = lax.rem(my + N_DEV - 1, N_DEV)
        right = lax.rem(my + 1, N_DEV)

        barrier = pltpu.get_barrier_semaphore()
        for nbr in (left, right):
            pl.semaphore_signal(
                barrier, inc=1, device_id=(nbr,),
                device_id_type=pl.DeviceIdType.MESH,
            )
        pl.semaphore_wait(barrier, 2)

        g_u[pl.ds(my, 1)] = u_ref[...][None]
        g_ml[pl.ds(my, 1), 0] = m_ref[:, 0, :][None]
        g_ml[pl.ds(my, 1), 1] = l_ref[:, 0, :][None]

        for h in range(N_DEV - 1):
            slot = lax.rem(my + N_DEV - h, N_DEV)
            rdma_u = pltpu.make_async_remote_copy(
                src_ref=g_u.at[pl.ds(slot, 1)],
                dst_ref=g_u.at[pl.ds(slot, 1)],
                send_sem=send_u.at[h],
                recv_sem=recv_u.at[h],
                device_id=(right,),
                device_id_type=pl.DeviceIdType.MESH,
            )
            rdma_ml = pltpu.make_async_remote_copy(
                src_ref=g_ml.at[pl.ds(slot, 1)],
                dst_ref=g_ml.at[pl.ds(slot, 1)],
                send_sem=send_ml.at[h],
                recv_sem=recv_ml.at[h],
                device_id=(right,),
                device_id_type=pl.DeviceIdType.MESH,
            )
            rdma_u.start()
            rdma_ml.start()
            rdma_u.wait()
            rdma_ml.wait()

        m_all = g_ml[:, 0]
        l_all = g_ml[:, 1]
        m_g = jnp.max(m_all, axis=0)
        alpha = jnp.exp(m_all - m_g[None])
        l_g = jnp.sum(l_all * alpha, axis=0)
        u_g = jnp.sum(g_u[...] * alpha[..., None], axis=0)
        out_ref[...] = (u_g / l_g[..., None])[:, None]

    return pl.pallas_call(
        body,
        out_shape=jax.ShapeDtypeStruct((B, SQ, H, D), jnp.float32),
        in_specs=[pl.BlockSpec(memory_space=pltpu.VMEM)] * 3,
        out_specs=pl.BlockSpec(memory_space=pltpu.VMEM),
        scratch_shapes=[
            pltpu.VMEM((N_DEV, B, H, D), jnp.float32),
            pltpu.VMEM((N_DEV, 2, B, H), jnp.float32),
            pltpu.SemaphoreType.DMA((N_DEV - 1,)),
            pltpu.SemaphoreType.DMA((N_DEV - 1,)),
            pltpu.SemaphoreType.DMA((N_DEV - 1,)),
            pltpu.SemaphoreType.DMA((N_DEV - 1,)),
        ],
        compiler_params=pltpu.CompilerParams(collective_id=0),
    )(U, m, l)


def kernel(Q, K, V):
    Kt = jnp.transpose(K, (0, 2, 3, 1))
    Vt = jnp.transpose(V, (0, 2, 3, 1))
    U, m, l = _local_partials(Q, Kt, Vt)
    return _ring_combine(U, m, l)


# device time: 53572 ns/iter; 6.0349x vs baseline; 1.0882x over previous
import functools

import jax
import jax.numpy as jnp
from jax import lax
from jax.experimental import pallas as pl
from jax.experimental.pallas import tpu as pltpu

N_DEV = 4
B, SQ, H, D = 16, 1, 16, 64
SCALE = D ** -0.5


def _local_partials(Q, Kt, Vt):
    kv_per = Kt.shape[3]
    hd = H * D

    def body(q_ref, k_ref, v_ref, u_ref, m_ref, l_ref):
        q = q_ref[0, 0]
        k2 = k_ref[0].reshape(hd, kv_per)
        v2 = v_ref[0].reshape(hd, kv_per)
        j = lax.broadcasted_iota(jnp.int32, (H, hd), 1)
        hh = lax.broadcasted_iota(jnp.int32, (H, hd), 0)
        A = jnp.where(j // D == hh, jnp.tile(q, (1, H)), 0.0)
        s = lax.dot_general(
            A, k2,
            dimension_numbers=(((1,), (0,)), ((), ())),
            preferred_element_type=jnp.float32,
        ) * SCALE
        m = jnp.max(s, axis=1)
        p = jnp.exp(s - m[:, None])
        l = jnp.sum(p, axis=1)
        p2 = jnp.broadcast_to(p[:, None, :], (H, D, kv_per)).reshape(hd, kv_per)
        ones_kv = jnp.ones((kv_per,), jnp.float32)
        u = lax.dot_general(
            v2 * p2, ones_kv,
            dimension_numbers=(((1,), (0,)), ((), ())),
            preferred_element_type=jnp.float32,
        ).reshape(H, D)
        u_ref[0] = u
        m_ref[0, 0, :] = m
        l_ref[0, 0, :] = l

    return pl.pallas_call(
        body,
        grid=(B,),
        in_specs=[
            pl.BlockSpec((1, SQ, H, D), lambda b: (b, 0, 0, 0)),
            pl.BlockSpec((1, H, D, kv_per), lambda b: (b, 0, 0, 0)),
            pl.BlockSpec((1, H, D, kv_per), lambda b: (b, 0, 0, 0)),
        ],
        out_specs=[
            pl.BlockSpec((1, H, D), lambda b: (b, 0, 0)),
            pl.BlockSpec((1, 1, H), lambda b: (b, 0, 0)),
            pl.BlockSpec((1, 1, H), lambda b: (b, 0, 0)),
        ],
        out_shape=[
            jax.ShapeDtypeStruct((B, H, D), jnp.float32),
            jax.ShapeDtypeStruct((B, 1, H), jnp.float32),
            jax.ShapeDtypeStruct((B, 1, H), jnp.float32),
        ],
        compiler_params=pltpu.CompilerParams(
            vmem_limit_bytes=100 * 1024 * 1024,
        ),
    )(Q, Kt, Vt)


def _ring_combine(U, m, l):

    def body(u_ref, m_ref, l_ref, out_ref, g_u, g_ml, send_u, recv_u,
             send_ml, recv_ml):
        my = lax.axis_index("i")

        barrier = pltpu.get_barrier_semaphore()
        for step in range(1, N_DEV):
            nbr = lax.rem(my + step, N_DEV)
            pl.semaphore_signal(
                barrier, inc=1, device_id=(nbr,),
                device_id_type=pl.DeviceIdType.MESH,
            )
        pl.semaphore_wait(barrier, N_DEV - 1)

        g_u[pl.ds(my, 1)] = u_ref[...][None]
        g_ml[pl.ds(my, 1), 0] = m_ref[:, 0, :][None]
        g_ml[pl.ds(my, 1), 1] = l_ref[:, 0, :][None]

        rdmas = []
        for step in range(1, N_DEV):
            peer = lax.rem(my + step, N_DEV)
            rdma_u = pltpu.make_async_remote_copy(
                src_ref=g_u.at[pl.ds(my, 1)],
                dst_ref=g_u.at[pl.ds(my, 1)],
                send_sem=send_u.at[step - 1],
                recv_sem=recv_u.at[my],
                device_id=(peer,),
                device_id_type=pl.DeviceIdType.MESH,
            )
            rdma_ml = pltpu.make_async_remote_copy(
                src_ref=g_ml.at[pl.ds(my, 1)],
                dst_ref=g_ml.at[pl.ds(my, 1)],
                send_sem=send_ml.at[step - 1],
                recv_sem=recv_ml.at[my],
                device_id=(peer,),
                device_id_type=pl.DeviceIdType.MESH,
            )
            rdma_u.start()
            rdma_ml.start()
            rdmas.append((rdma_u, rdma_ml))

        for step, (rdma_u, rdma_ml) in zip(range(1, N_DEV), rdmas):
            peer = lax.rem(my + step, N_DEV)
            rdma_u.wait_send()
            rdma_ml.wait_send()
            recv_wait_u = pltpu.make_async_remote_copy(
                src_ref=g_u.at[pl.ds(my, 1)],
                dst_ref=g_u.at[pl.ds(peer, 1)],
                send_sem=send_u.at[step - 1],
                recv_sem=recv_u.at[peer],
                device_id=(peer,),
                device_id_type=pl.DeviceIdType.MESH,
            )
            recv_wait_ml = pltpu.make_async_remote_copy(
                src_ref=g_ml.at[pl.ds(my, 1)],
                dst_ref=g_ml.at[pl.ds(peer, 1)],
                send_sem=send_ml.at[step - 1],
                recv_sem=recv_ml.at[peer],
                device_id=(peer,),
                device_id_type=pl.DeviceIdType.MESH,
            )
            recv_wait_u.wait_recv()
            recv_wait_ml.wait_recv()

        m_all = g_ml[:, 0]
        l_all = g_ml[:, 1]
        m_g = jnp.max(m_all, axis=0)
        alpha = jnp.exp(m_all - m_g[None])
        l_g = jnp.sum(l_all * alpha, axis=0)
        u_g = jnp.sum(g_u[...] * alpha[..., None], axis=0)
        out_ref[...] = (u_g / l_g[..., None])[:, None]

    return pl.pallas_call(
        body,
        out_shape=jax.ShapeDtypeStruct((B, SQ, H, D), jnp.float32),
        in_specs=[pl.BlockSpec(memory_space=pltpu.VMEM)] * 3,
        out_specs=pl.BlockSpec(memory_space=pltpu.VMEM),
        scratch_shapes=[
            pltpu.VMEM((N_DEV, B, H, D), jnp.float32),
            pltpu.VMEM((N_DEV, 2, B, H), jnp.float32),
            pltpu.SemaphoreType.DMA((N_DEV - 1,)),
            pltpu.SemaphoreType.DMA((N_DEV,)),
            pltpu.SemaphoreType.DMA((N_DEV - 1,)),
            pltpu.SemaphoreType.DMA((N_DEV,)),
        ],
        compiler_params=pltpu.CompilerParams(collective_id=0),
    )(U, m, l)


def kernel(Q, K, V):
    Kt = jnp.transpose(K, (0, 2, 3, 1))
    Vt = jnp.transpose(V, (0, 2, 3, 1))
    U, m, l = _local_partials(Q, Kt, Vt)
    return _ring_combine(U, m, l)


# device time: 49505 ns/iter; 6.5306x vs baseline; 1.0822x over previous
import functools

import jax
import jax.numpy as jnp
from jax import lax
from jax.experimental import pallas as pl
from jax.experimental.pallas import tpu as pltpu

N_DEV = 4
B, SQ, H, D = 16, 1, 16, 64
SCALE = D ** -0.5


def _local_partials(Q, Kt, Vt):
    kv_per = Kt.shape[3]
    hd = H * D

    def body(q_ref, k_ref, v_ref, u_ref, m_ref, l_ref):
        q = q_ref[0, 0]
        k2 = k_ref[0].reshape(hd, kv_per)
        v2 = v_ref[0].reshape(hd, kv_per)
        j = lax.broadcasted_iota(jnp.int32, (H, hd), 1)
        hh = lax.broadcasted_iota(jnp.int32, (H, hd), 0)
        A = jnp.where(j // D == hh, jnp.tile(q, (1, H)), 0.0)
        s = lax.dot_general(
            A, k2,
            dimension_numbers=(((1,), (0,)), ((), ())),
            preferred_element_type=jnp.float32,
        ) * SCALE
        m = jnp.max(s, axis=1)
        p = jnp.exp(s - m[:, None])
        l = jnp.sum(p, axis=1)
        p2 = jnp.broadcast_to(p[:, None, :], (H, D, kv_per)).reshape(hd, kv_per)
        ones_kv = jnp.ones((kv_per,), jnp.float32)
        u = lax.dot_general(
            v2 * p2, ones_kv,
            dimension_numbers=(((1,), (0,)), ((), ())),
            preferred_element_type=jnp.float32,
        ).reshape(H, D)
        u_ref[0] = u
        m_ref[0, 0, :] = m
        l_ref[0, 0, :] = l

    return pl.pallas_call(
        body,
        grid=(B,),
        in_specs=[
            pl.BlockSpec((1, SQ, H, D), lambda b: (b, 0, 0, 0)),
            pl.BlockSpec((1, H, D, kv_per), lambda b: (b, 0, 0, 0)),
            pl.BlockSpec((1, H, D, kv_per), lambda b: (b, 0, 0, 0)),
        ],
        out_specs=[
            pl.BlockSpec((1, H, D), lambda b: (b, 0, 0)),
            pl.BlockSpec((1, 1, H), lambda b: (b, 0, 0)),
            pl.BlockSpec((1, 1, H), lambda b: (b, 0, 0)),
        ],
        out_shape=[
            jax.ShapeDtypeStruct((B, H, D), jnp.float32),
            jax.ShapeDtypeStruct((B, 1, H), jnp.float32),
            jax.ShapeDtypeStruct((B, 1, H), jnp.float32),
        ],
        compiler_params=pltpu.CompilerParams(
            vmem_limit_bytes=100 * 1024 * 1024,
        ),
    )(Q, Kt, Vt)


def _ring_combine(U, m, l):

    def body(u_ref, m_ref, l_ref, out_ref, g_u, g_ml, send_u, recv_u,
             send_ml, recv_ml):
        my = lax.axis_index("i")

        barrier = pltpu.get_barrier_semaphore()
        for step in range(1, N_DEV):
            nbr = lax.rem(my + step, N_DEV)
            pl.semaphore_signal(
                barrier, inc=1, device_id=(nbr,),
                device_id_type=pl.DeviceIdType.MESH,
            )
        pl.semaphore_wait(barrier, N_DEV - 1)

        g_u[pl.ds(my, 1)] = u_ref[...][None]
        g_ml[pl.ds(my, 1), 0] = m_ref[:, 0, :][None]
        g_ml[pl.ds(my, 1), 1] = l_ref[:, 0, :][None]

        rdmas = []
        for step in range(1, N_DEV):
            peer = lax.rem(my + step, N_DEV)
            rdma_u = pltpu.make_async_remote_copy(
                src_ref=g_u.at[pl.ds(my, 1)],
                dst_ref=g_u.at[pl.ds(my, 1)],
                send_sem=send_u.at[step - 1],
                recv_sem=recv_u.at[my],
                device_id=(peer,),
                device_id_type=pl.DeviceIdType.MESH,
            )
            rdma_ml = pltpu.make_async_remote_copy(
                src_ref=g_ml.at[pl.ds(my, 1)],
                dst_ref=g_ml.at[pl.ds(my, 1)],
                send_sem=send_ml.at[step - 1],
                recv_sem=recv_ml.at[my],
                device_id=(peer,),
                device_id_type=pl.DeviceIdType.MESH,
            )
            rdma_u.start()
            rdma_ml.start()
            rdmas.append((rdma_u, rdma_ml))

        for step, (rdma_u, rdma_ml) in zip(range(1, N_DEV), rdmas):
            peer = lax.rem(my + step, N_DEV)
            rdma_u.wait_send()
            rdma_ml.wait_send()
            recv_wait_u = pltpu.make_async_remote_copy(
                src_ref=g_u.at[pl.ds(my, 1)],
                dst_ref=g_u.at[pl.ds(peer, 1)],
                send_sem=send_u.at[step - 1],
                recv_sem=recv_u.at[peer],
                device_id=(peer,),
                device_id_type=pl.DeviceIdType.MESH,
            )
            recv_wait_ml = pltpu.make_async_remote_copy(
                src_ref=g_ml.at[pl.ds(my, 1)],
                dst_ref=g_ml.at[pl.ds(peer, 1)],
                send_sem=send_ml.at[step - 1],
                recv_sem=recv_ml.at[peer],
                device_id=(peer,),
                device_id_type=pl.DeviceIdType.MESH,
            )
            recv_wait_u.wait_recv()
            recv_wait_ml.wait_recv()

        m_all = g_ml[:, 0]
        l_all = g_ml[:, 1]
        m_g = jnp.max(m_all, axis=0)
        alpha = jnp.exp(m_all - m_g[None])
        l_g = jnp.sum(l_all * alpha, axis=0)
        u_g = jnp.sum(g_u[...] * alpha[..., None], axis=0)
        out_ref[...] = (u_g / l_g[..., None])[:, None]

    return pl.pallas_call(
        body,
        out_shape=jax.ShapeDtypeStruct((B, SQ, H, D), jnp.float32),
        in_specs=[pl.BlockSpec(memory_space=pltpu.VMEM)] * 3,
        out_specs=pl.BlockSpec(memory_space=pltpu.VMEM),
        scratch_shapes=[
            pltpu.VMEM((N_DEV, B, H, D), jnp.float32),
            pltpu.VMEM((N_DEV, 2, B, H), jnp.float32),
            pltpu.SemaphoreType.DMA((N_DEV - 1,)),
            pltpu.SemaphoreType.DMA((N_DEV,)),
            pltpu.SemaphoreType.DMA((N_DEV - 1,)),
            pltpu.SemaphoreType.DMA((N_DEV,)),
        ],
        compiler_params=pltpu.CompilerParams(collective_id=0),
    )(U, m, l)


def _fused(Q, Kt, Vt):
    kv_per = Kt.shape[3]
    hd = H * D

    def body(q_ref, k_ref, v_ref, out_ref, g, send_sems, recv_sems):
        my = lax.axis_index("i")
        b = pl.program_id(0)

        @pl.when(b == 0)
        def _():
            barrier = pltpu.get_barrier_semaphore()
            for step in range(1, N_DEV):
                nbr = lax.rem(my + step, N_DEV)
                pl.semaphore_signal(
                    barrier, inc=1, device_id=(nbr,),
                    device_id_type=pl.DeviceIdType.MESH,
                )
            pl.semaphore_wait(barrier, N_DEV - 1)

        q = q_ref[0, 0]
        k2 = k_ref[0].reshape(hd, kv_per)
        v2 = v_ref[0].reshape(hd, kv_per)
        j = lax.broadcasted_iota(jnp.int32, (H, hd), 1)
        hh = lax.broadcasted_iota(jnp.int32, (H, hd), 0)
        A = jnp.where(j // D == hh, jnp.tile(q, (1, H)), 0.0)
        s = lax.dot_general(
            A, k2,
            dimension_numbers=(((1,), (0,)), ((), ())),
            preferred_element_type=jnp.float32,
        ) * SCALE
        m = jnp.max(s, axis=1)
        p = jnp.exp(s - m[:, None])
        l = jnp.sum(p, axis=1)
        p2 = jnp.broadcast_to(p[:, None, :], (H, D, kv_per)).reshape(hd, kv_per)
        ones_kv = jnp.ones((kv_per,), jnp.float32)
        u = lax.dot_general(
            v2 * p2, ones_kv,
            dimension_numbers=(((1,), (0,)), ((), ())),
            preferred_element_type=jnp.float32,
        ).reshape(H, D)

        g[my, b, :, 0:D] = u
        g[my, b, :, D:D + 1] = m[:, None]
        g[my, b, :, D + 1:D + 2] = l[:, None]
        for step in range(1, N_DEV):
            peer = lax.rem(my + step, N_DEV)
            pltpu.make_async_remote_copy(
                src_ref=g.at[my, b],
                dst_ref=g.at[my, b],
                send_sem=send_sems.at[step - 1, b],
                recv_sem=recv_sems.at[my, b],
                device_id=(peer,),
                device_id_type=pl.DeviceIdType.MESH,
            ).start()

        @pl.when(b == B - 1)
        def _():
            for bb in range(B):
                for step in range(1, N_DEV):
                    peer = lax.rem(my + step, N_DEV)
                    done = pltpu.make_async_remote_copy(
                        src_ref=g.at[my, bb],
                        dst_ref=g.at[peer, bb],
                        send_sem=send_sems.at[step - 1, bb],
                        recv_sem=recv_sems.at[peer, bb],
                        device_id=(peer,),
                        device_id_type=pl.DeviceIdType.MESH,
                    )
                    done.wait_send()
                    done.wait_recv()

            m_all = g[:, :, :, D]
            l_all = g[:, :, :, D + 1]
            m_g = jnp.max(m_all, axis=0)
            alpha = jnp.exp(m_all - m_g[None])
            l_g = jnp.sum(l_all * alpha, axis=0)
            u_g = jnp.sum(g[:, :, :, 0:D] * alpha[..., None], axis=0)
            out_ref[...] = (u_g / l_g[..., None])[:, None]

    return pl.pallas_call(
        body,
        grid=(B,),
        in_specs=[
            pl.BlockSpec((1, SQ, H, D), lambda b: (b, 0, 0, 0)),
            pl.BlockSpec((1, H, D, kv_per), lambda b: (b, 0, 0, 0)),
            pl.BlockSpec((1, H, D, kv_per), lambda b: (b, 0, 0, 0)),
        ],
        out_specs=pl.BlockSpec((B, SQ, H, D), lambda b: (0, 0, 0, 0)),
        out_shape=jax.ShapeDtypeStruct((B, SQ, H, D), jnp.float32),
        scratch_shapes=[
            pltpu.VMEM((N_DEV, B, H, 128), jnp.float32),
            pltpu.SemaphoreType.DMA((N_DEV - 1, B)),
            pltpu.SemaphoreType.DMA((N_DEV, B)),
        ],
        compiler_params=pltpu.CompilerParams(
            collective_id=0,
            vmem_limit_bytes=100 * 1024 * 1024,
        ),
    )(Q, Kt, Vt)


def kernel(Q, K, V):
    Kt = jnp.transpose(K, (0, 2, 3, 1))
    Vt = jnp.transpose(V, (0, 2, 3, 1))
    return _fused(Q, Kt, Vt)
